# baseline (speedup 1.0000x reference)
import functools
import math

import jax
import jax.numpy as jnp
from jax import lax
from jax.experimental import pallas as pl
from jax.experimental.pallas import tpu as pltpu
from jax.experimental.pallas import tpu_sc as plsc

N_NODES = 100000
N_EDGES = 1600000
NUM_RBF = 64
NF = NUM_RBF + 1
MAX_DIST = 25.0

_NC = 2
_NS = 16
_NW = _NC * _NS
_EPW = N_EDGES // _NW
_CHUNK = 2000
_NCHUNK = _EPW // _CHUNK
_NPAD = 102400
_STRIPE = _NPAD // _NS


def _sc_gather(px, py, pz, dest, src,
               pdx, pdy, pdz, psx, psy, psz,
               plane, didx, sidx, outd, outs):
    cid = lax.axis_index("c")
    sid = lax.axis_index("s")
    wid = cid * _NS + sid

    def do_chunk(base):
        pltpu.sync_copy(dest.at[pl.ds(base, _CHUNK)], didx)
        pltpu.sync_copy(src.at[pl.ds(base, _CHUNK)], sidx)

        def body(j, _):
            s = pl.multiple_of(j * 16, 8)
            gd = plsc.load_gather(plane, [didx[pl.ds(s, 16)]])
            gs = plsc.load_gather(plane, [sidx[pl.ds(s, 16)]])
            outd[pl.ds(s, 16)] = gd
            outs[pl.ds(s, 16)] = gs
            return ()

        lax.fori_loop(0, _CHUNK // 16, body, ())

    for hbm_plane, od_arr, os_arr in ((px, pdx, psx), (py, pdy, psy),
                                      (pz, pdz, psz)):
        pltpu.sync_copy(hbm_plane, plane)
        for i in range(_NCHUNK):
            base = pl.multiple_of(wid * _EPW + i * _CHUNK, 8)
            do_chunk(base)
            pltpu.sync_copy(outd, od_arr.at[pl.ds(base, _CHUNK)])
            pltpu.sync_copy(outs, os_arr.at[pl.ds(base, _CHUNK)])


_sc_gather_call = functools.partial(
    pl.kernel,
    mesh=plsc.VectorSubcoreMesh(core_axis_name="c", subcore_axis_name="s"),
    out_type=[jax.ShapeDtypeStruct((N_EDGES,), jnp.float32)] * 6,
    scratch_types=[
        pltpu.VMEM((_NPAD,), jnp.float32),
        pltpu.VMEM((_CHUNK,), jnp.int32),
        pltpu.VMEM((_CHUNK,), jnp.int32),
        pltpu.VMEM((_CHUNK,), jnp.float32),
        pltpu.VMEM((_CHUNK,), jnp.float32),
    ],
    compiler_params=pltpu.CompilerParams(needs_layout_passes=False),
)(_sc_gather)


_BT = 25600


def _tc_featurize(pdx_r, pdy_r, pdz_r, psx_r, psy_r, psz_r, off_r, invd_r,
                  featT_ref, vecT_ref):
    ax, ay, az = pdx_r[...], pdy_r[...], pdz_r[...]
    bx, by, bz = psx_r[...], psy_r[...], psz_r[...]
    vx, vy, vz = ax - bx, ay - by, az - bz
    d2 = vx * vx + vy * vy + vz * vz + 1e-6
    d = jnp.sqrt(d2)
    num = ax * bx + ay * by + az * bz
    a2 = ax * ax + ay * ay + az * az
    b2 = bx * bx + by * by + bz * bz
    cosv = num * lax.rsqrt(jnp.maximum(a2, 1e-16) * jnp.maximum(b2, 1e-16))
    dc = jnp.minimum(d, MAX_DIST)
    cut = jnp.where(dc < MAX_DIST,
                    0.5 * (jnp.cos(dc * (math.pi / MAX_DIST)) + 1.0), 0.0)
    t = dc[None, :] - off_r[...]
    feats = jnp.exp(-(t * t) * invd_r[...])
    feats = feats * cut[None, :]
    row = lax.broadcasted_iota(jnp.int32, (NF, _BT), 0)
    featT_ref[...] = jnp.where(row == NUM_RBF, cosv[None, :], feats)
    r = 1.0 / (d + 1.0)
    vecT_ref[...] = jnp.concatenate(
        [(vx * r)[None, :], (vy * r)[None, :], (vz * r)[None, :]], axis=0)


def kernel(positions, edge_index, offsets, delta):
    pad = jnp.zeros((_NPAD - N_NODES,), jnp.float32)
    px = jnp.concatenate([positions[:, 0], pad])
    py = jnp.concatenate([positions[:, 1], pad])
    pz = jnp.concatenate([positions[:, 2], pad])
    dest, src = edge_index[0], edge_index[1]
    pdx, pdy, pdz, psx, psy, psz = _sc_gather_call(px, py, pz, dest, src)

    offc = jnp.concatenate(
        [offsets.reshape(NUM_RBF), jnp.zeros((1,), jnp.float32)]).reshape(NF, 1)
    invd = (1.0 / delta).astype(jnp.float32).reshape(1, 1)

    grid = ((N_EDGES + _BT - 1) // _BT,)
    espec = pl.BlockSpec((_BT,), lambda i: (i,))
    featT, vecT = pl.pallas_call(
        _tc_featurize,
        grid=grid,
        in_specs=[espec] * 6 + [
            pl.BlockSpec((NF, 1), lambda i: (0, 0)),
            pl.BlockSpec((1, 1), lambda i: (0, 0)),
        ],
        out_specs=[
            pl.BlockSpec((NF, _BT), lambda i: (0, i)),
            pl.BlockSpec((3, _BT), lambda i: (0, i)),
        ],
        out_shape=[
            jax.ShapeDtypeStruct((NF, N_EDGES), jnp.float32),
            jax.ShapeDtypeStruct((3, N_EDGES), jnp.float32),
        ],
    )(pdx, pdy, pdz, psx, psy, psz, offc, invd)
    return (featT.T, vecT.T)


# speedup vs baseline: 1.1776x; 1.1776x over previous
import functools
import math

import jax
import jax.numpy as jnp
from jax import lax
from jax.experimental import pallas as pl
from jax.experimental.pallas import tpu as pltpu
from jax.experimental.pallas import tpu_sc as plsc

N_NODES = 100000
N_EDGES = 1600000
NUM_RBF = 64
NF = NUM_RBF + 1
MAX_DIST = 25.0

_NC = 2
_NS = 16
_NW = _NC * _NS
_EPW = N_EDGES // _NW
_CHUNK = 10000
_NCHUNK = _EPW // _CHUNK
_NPAD = 102400
_STRIPE = _NPAD // _NS


def _sc_gather(px, py, pz, dest, src,
               pdx, pdy, pdz, psx, psy, psz,
               spx, spy, spz, stg, didx, sidx, g, sems):
    cid = lax.axis_index("c")
    sid = lax.axis_index("s")
    soff = pl.multiple_of(sid * _STRIPE, 8)
    for hbm, sp in ((px, spx), (py, spy), (pz, spz)):
        pltpu.sync_copy(hbm.at[pl.ds(soff, _STRIPE)], stg)
        pltpu.sync_copy(stg, sp.at[pl.ds(soff, _STRIPE)])
    plsc.subcore_barrier()

    wid = cid * _NS + sid
    for i in range(_NCHUNK):
        base = pl.multiple_of(wid * _EPW + i * _CHUNK, 8)
        pltpu.sync_copy(dest.at[pl.ds(base, _CHUNK)], didx)
        pltpu.sync_copy(src.at[pl.ds(base, _CHUNK)], sidx)
        cps = [
            pltpu.async_copy(spx.at[didx], g[0], sems[0]),
            pltpu.async_copy(spy.at[didx], g[1], sems[1]),
            pltpu.async_copy(spz.at[didx], g[2], sems[2]),
            pltpu.async_copy(spx.at[sidx], g[3], sems[3]),
            pltpu.async_copy(spy.at[sidx], g[4], sems[4]),
            pltpu.async_copy(spz.at[sidx], g[5], sems[5]),
        ]
        for cp in cps:
            cp.wait()
        for buf, out in zip(g, (pdx, pdy, pdz, psx, psy, psz)):
            pltpu.sync_copy(buf, out.at[pl.ds(base, _CHUNK)])


_sc_gather_call = functools.partial(
    pl.kernel,
    mesh=plsc.VectorSubcoreMesh(core_axis_name="c", subcore_axis_name="s"),
    out_type=[jax.ShapeDtypeStruct((N_EDGES,), jnp.float32)] * 6,
    scratch_types=[
        pltpu.VMEM_SHARED((_NPAD,), jnp.float32),
        pltpu.VMEM_SHARED((_NPAD,), jnp.float32),
        pltpu.VMEM_SHARED((_NPAD,), jnp.float32),
        pltpu.VMEM((_STRIPE,), jnp.float32),
        pltpu.VMEM((_CHUNK,), jnp.int32),
        pltpu.VMEM((_CHUNK,), jnp.int32),
        [pltpu.VMEM((_CHUNK,), jnp.float32)] * 6,
        [pltpu.SemaphoreType.DMA] * 6,
    ],
)(_sc_gather)


_BT = 51200


def _tc_featurize(pdx_r, pdy_r, pdz_r, psx_r, psy_r, psz_r, off_r, invd_r,
                  featT_ref, vecT_ref):
    ax, ay, az = pdx_r[...], pdy_r[...], pdz_r[...]
    bx, by, bz = psx_r[...], psy_r[...], psz_r[...]
    vx, vy, vz = ax - bx, ay - by, az - bz
    d2 = vx * vx + vy * vy + vz * vz + 1e-6
    d = jnp.sqrt(d2)
    num = ax * bx + ay * by + az * bz
    a2 = ax * ax + ay * ay + az * az
    b2 = bx * bx + by * by + bz * bz
    cosv = num * lax.rsqrt(jnp.maximum(a2, 1e-16) * jnp.maximum(b2, 1e-16))
    dc = jnp.minimum(d, MAX_DIST)
    cut = jnp.where(dc < MAX_DIST,
                    0.5 * (jnp.cos(dc * (math.pi / MAX_DIST)) + 1.0), 0.0)
    t = dc[None, :] - off_r[...]
    feats = jnp.exp(-(t * t) * invd_r[...])
    feats = feats * cut[None, :]
    row = lax.broadcasted_iota(jnp.int32, (NF, _BT), 0)
    featT_ref[...] = jnp.where(row == NUM_RBF, cosv[None, :], feats)
    r = 1.0 / (d + 1.0)
    vecT_ref[...] = jnp.concatenate(
        [(vx * r)[None, :], (vy * r)[None, :], (vz * r)[None, :]], axis=0)


def kernel(positions, edge_index, offsets, delta):
    pad = jnp.zeros((_NPAD - N_NODES,), jnp.float32)
    px = jnp.concatenate([positions[:, 0], pad])
    py = jnp.concatenate([positions[:, 1], pad])
    pz = jnp.concatenate([positions[:, 2], pad])
    dest, src = edge_index[0], edge_index[1]
    pdx, pdy, pdz, psx, psy, psz = _sc_gather_call(px, py, pz, dest, src)

    offc = jnp.concatenate(
        [offsets.reshape(NUM_RBF), jnp.zeros((1,), jnp.float32)]).reshape(NF, 1)
    invd = (1.0 / delta).astype(jnp.float32).reshape(1, 1)

    grid = ((N_EDGES + _BT - 1) // _BT,)
    espec = pl.BlockSpec((_BT,), lambda i: (i,))
    featT, vecT = pl.pallas_call(
        _tc_featurize,
        grid=grid,
        in_specs=[espec] * 6 + [
            pl.BlockSpec((NF, 1), lambda i: (0, 0)),
            pl.BlockSpec((1, 1), lambda i: (0, 0)),
        ],
        out_specs=[
            pl.BlockSpec((NF, _BT), lambda i: (0, i)),
            pl.BlockSpec((3, _BT), lambda i: (0, i)),
        ],
        out_shape=[
            jax.ShapeDtypeStruct((NF, N_EDGES), jnp.float32),
            jax.ShapeDtypeStruct((3, N_EDGES), jnp.float32),
        ],
    )(pdx, pdy, pdz, psx, psy, psz, offc, invd)
    return (featT.T, vecT.T)
